# Initial kernel scaffold; baseline (speedup 1.0000x reference)
import jax
import jax.numpy as jnp
from jax.experimental import pallas as pl


def kernel(z, embedding):
    raise NotImplementedError("write your pallas kernel here")


# speedup vs baseline: 1.1024x; 1.1024x over previous
import functools

import jax
import jax.numpy as jnp
from jax import lax
from jax.experimental import pallas as pl
from jax.experimental.pallas import tpu as pltpu
from jax.experimental.pallas import tpu_sc as plsc

M_BLK = 512
K_BLK = 2048


def _argmin_body(z_ref, e_ref, idx_ref, mval_ref, midx_ref):
    k = pl.program_id(1)
    nk = pl.num_programs(1)
    z = z_ref[...]
    e = e_ref[...]
    mm = lax.dot_general(z, e, (((1,), (1,)), ((), ())),
                         preferred_element_type=jnp.float32)
    zn = jnp.sum(z * z, axis=1, keepdims=True)
    en = jnp.sum(e * e, axis=1)
    d = zn + en[None, :] - 2.0 * mm
    m = jnp.min(d, axis=1, keepdims=True)
    cols = lax.broadcasted_iota(jnp.int32, d.shape, 1) + k * K_BLK
    a = jnp.min(jnp.where(d == m, cols, jnp.int32(2**30)),
                axis=1, keepdims=True)

    @pl.when(k == 0)
    def _():
        mval_ref[...] = m
        midx_ref[...] = a

    @pl.when(k > 0)
    def _():
        better = m < mval_ref[...]
        mval_ref[...] = jnp.where(better, m, mval_ref[...])
        midx_ref[...] = jnp.where(better, a, midx_ref[...])

    @pl.when(k == nk - 1)
    def _():
        idx_ref[...] = midx_ref[...][:, 0]


def _argmin_call(z_flat, embedding):
    n, d = z_flat.shape
    n_e = embedding.shape[0]
    return pl.pallas_call(
        _argmin_body,
        grid=(n // M_BLK, n_e // K_BLK),
        in_specs=[
            pl.BlockSpec((M_BLK, d), lambda i, k: (i, 0)),
            pl.BlockSpec((K_BLK, d), lambda i, k: (k, 0)),
        ],
        out_specs=pl.BlockSpec((M_BLK,), lambda i, k: (i,)),
        out_shape=jax.ShapeDtypeStruct((n,), jnp.int32),
        scratch_shapes=[
            pltpu.VMEM((M_BLK, 1), jnp.float32),
            pltpu.VMEM((M_BLK, 1), jnp.int32),
        ],
        compiler_params=pltpu.CompilerParams(
            dimension_semantics=("parallel", "arbitrary")),
    )(z_flat, embedding)


@functools.cache
def _make_sc_gather(v, d, b):
    info = plsc.get_sparse_core_info()
    nc, ns = info.num_cores, info.num_subcores
    nw = nc * ns
    assert d % info.num_lanes == 0 and b % (8 * nw) == 0
    b_per_w = b // nw
    mesh = plsc.VectorSubcoreMesh(core_axis_name="c", subcore_axis_name="s")

    @functools.partial(
        pl.kernel, mesh=mesh,
        out_type=jax.ShapeDtypeStruct((b, d), jnp.float32),
        scratch_types=[
            pltpu.VMEM((b_per_w,), jnp.int32),
            pltpu.VMEM((b_per_w, d), jnp.float32),
            pltpu.SemaphoreType.DMA,
        ],
    )
    def gather(table_hbm, idx_hbm, out_hbm, idx_v, rows_v, sem):
        wid = lax.axis_index("s") * nc + lax.axis_index("c")
        base = wid * b_per_w
        pltpu.sync_copy(idx_hbm.at[pl.ds(base, b_per_w)], idx_v)
        pltpu.async_copy(table_hbm.at[idx_v], rows_v, sem).wait()
        pltpu.sync_copy(rows_v, out_hbm.at[pl.ds(base, b_per_w)])

    return gather


def kernel(z, embedding):
    bsz, c, h, w = z.shape
    zp = jnp.transpose(z, (0, 2, 3, 1))
    z_flat = zp.reshape(-1, c)
    idx = _argmin_call(z_flat, embedding)
    zq_flat = _make_sc_gather(embedding.shape[0], c, z_flat.shape[0])(
        embedding, idx)
    z_q = jnp.transpose(zq_flat.reshape(bsz, h, w, c), (0, 3, 1, 2))
    return (z_q, idx)
